# Initial kernel scaffold; baseline (speedup 1.0000x reference)
import jax
import jax.numpy as jnp
from jax.experimental import pallas as pl


def kernel(latents, embedding):
    raise NotImplementedError("write your pallas kernel here")


# speedup vs baseline: 1.5813x; 1.5813x over previous
import functools

import jax
import jax.numpy as jnp
from jax import lax
from jax.experimental import pallas as pl
from jax.experimental.pallas import tpu as pltpu
from jax.experimental.pallas import tpu_sc as plsc

K = 8192
D = 32
N = 32768
BETA = 0.25

RB = 1024
KC = 2048
N_CHUNKS = K // KC

SC_CORES = 2
SC_SUBCORES = 16
SC_WORKERS = SC_CORES * SC_SUBCORES
ROWS_PER_WORKER = N // SC_WORKERS
IDX_MINOR = 128
IDX_ROWS_PER_WORKER = ROWS_PER_WORKER // IDX_MINOR


def _argmin_body(flat_ref, embt2_ref, idx_ref, loss_ref):
    f = flat_ref[...]
    sumf = jnp.sum(f * f, axis=1, keepdims=True)

    bestd = jnp.full((RB, 1), jnp.inf, dtype=jnp.float32)
    besti = jnp.full((RB, 1), jnp.int32(0x7FFFFFFF), dtype=jnp.int32)
    for j in range(N_CHUNKS):
        m2 = jnp.dot(f, embt2_ref[:, j * KC:(j + 1) * KC],
                     preferred_element_type=jnp.float32)
        dist = sumf - m2
        cmin = jnp.min(dist, axis=1, keepdims=True)
        col = lax.broadcasted_iota(jnp.int32, (RB, KC), 1) + jnp.int32(j * KC)
        cidx = jnp.min(jnp.where(dist == cmin, col, jnp.int32(0x7FFFFFFF)),
                       axis=1, keepdims=True)
        upd = cmin < bestd
        bestd = jnp.where(upd, cmin, bestd)
        besti = jnp.where(upd, cidx, besti)

    idx_ref[...] = besti
    i = pl.program_id(0)

    @pl.when(i == 0)
    def _init():
        loss_ref[...] = jnp.zeros_like(loss_ref)

    loss_ref[...] += jnp.sum(bestd).reshape(1, 1)


_argmin_call = pl.pallas_call(
    _argmin_body,
    grid=(N // RB,),
    in_specs=[
        pl.BlockSpec((RB, D), lambda i: (i, 0)),
        pl.BlockSpec((D, K), lambda i: (0, 0)),
    ],
    out_specs=[
        pl.BlockSpec((RB, 1), lambda i: (i, 0)),
        pl.BlockSpec((1, 1), lambda i: (0, 0)),
    ],
    out_shape=[
        jax.ShapeDtypeStruct((N, 1), jnp.int32),
        jax.ShapeDtypeStruct((1, 1), jnp.float32),
    ],
    compiler_params=pltpu.CompilerParams(
        dimension_semantics=("arbitrary",),
    ),
)


def _gather_body(emb_hbm, idx_hbm, out_hbm, idx_v, rows_v, sem):
    c = lax.axis_index("c")
    s = lax.axis_index("s")
    wid = s * SC_CORES + c
    pltpu.sync_copy(idx_hbm.at[pl.ds(wid * IDX_ROWS_PER_WORKER,
                                     IDX_ROWS_PER_WORKER)], idx_v)
    copies = []
    for j in range(IDX_ROWS_PER_WORKER):
        copies.append(pltpu.async_copy(
            emb_hbm.at[idx_v.at[j]],
            rows_v.at[pl.ds(j * IDX_MINOR, IDX_MINOR)],
            sem))
    for cp in copies:
        cp.wait()
    pltpu.sync_copy(rows_v, out_hbm.at[pl.ds(wid * ROWS_PER_WORKER,
                                             ROWS_PER_WORKER)])


_gather_call = pl.kernel(
    _gather_body,
    out_type=jax.ShapeDtypeStruct((N, D), jnp.float32),
    mesh=plsc.VectorSubcoreMesh(core_axis_name="c", subcore_axis_name="s"),
    scratch_types=[
        pltpu.VMEM((IDX_ROWS_PER_WORKER, IDX_MINOR), jnp.int32),
        pltpu.VMEM((ROWS_PER_WORKER, D), jnp.float32),
        pltpu.SemaphoreType.DMA,
    ],
    compiler_params=pltpu.CompilerParams(use_tc_tiling_on_sc=False),
)


def kernel(latents, embedding):
    flat = jnp.transpose(latents, (0, 2, 3, 1)).reshape(-1, D)
    embt2 = jnp.transpose(embedding * jnp.float32(2.0))

    idx, loss_sum = _argmin_call(flat, embt2)

    q = _gather_call(embedding, idx.reshape(N // IDX_MINOR, IDX_MINOR))

    out = jnp.transpose(q.reshape(32, 32, 32, D), (0, 3, 1, 2))
    vq_loss = loss_sum[0, 0] * jnp.float32((1.0 + BETA) / (N * D))
    return out, vq_loss


# speedup vs baseline: 1.7899x; 1.1320x over previous
import functools

import jax
import jax.numpy as jnp
from jax import lax
from jax.experimental import pallas as pl
from jax.experimental.pallas import tpu as pltpu
from jax.experimental.pallas import tpu_sc as plsc

K = 8192
D = 32
N = 32768
BETA = 0.25

RB = 1024
KC = 2048
N_CHUNKS = K // KC

SC_CORES = 2
SC_SUBCORES = 16
SC_WORKERS = SC_CORES * SC_SUBCORES
ROWS_PER_WORKER = N // SC_WORKERS
IDX_MINOR = 128
IDX_ROWS_PER_WORKER = ROWS_PER_WORKER // IDX_MINOR


def _argmin_body(flat_ref, embt2_ref, idx_ref, loss_ref):
    f = flat_ref[...]
    sumf = jnp.sum(f * f, axis=1, keepdims=True)

    bestd = jnp.full((RB, 1), jnp.inf, dtype=jnp.float32)
    besti = jnp.full((RB, 1), jnp.float32(2.0**30), dtype=jnp.float32)
    colf = lax.broadcasted_iota(jnp.int32, (1, KC), 1).astype(jnp.float32)
    for j in range(N_CHUNKS):
        m2 = jnp.dot(f, embt2_ref[:, j * KC:(j + 1) * KC],
                     preferred_element_type=jnp.float32)
        dist = sumf - m2
        cmin = jnp.min(dist, axis=1, keepdims=True)
        cidx = jnp.min(jnp.where(dist == cmin, colf, jnp.float32(2.0**30)),
                       axis=1, keepdims=True) + jnp.float32(j * KC)
        upd = cmin < bestd
        bestd = jnp.where(upd, cmin, bestd)
        besti = jnp.where(upd, cidx, besti)

    idx_ref[...] = besti.astype(jnp.int32)
    i = pl.program_id(0)

    @pl.when(i == 0)
    def _init():
        loss_ref[...] = jnp.zeros_like(loss_ref)

    loss_ref[...] += jnp.sum(bestd).reshape(1, 1)


_argmin_call = pl.pallas_call(
    _argmin_body,
    grid=(N // RB,),
    in_specs=[
        pl.BlockSpec((RB, D), lambda i: (i, 0)),
        pl.BlockSpec((D, K), lambda i: (0, 0)),
    ],
    out_specs=[
        pl.BlockSpec((RB, 1), lambda i: (i, 0)),
        pl.BlockSpec((1, 1), lambda i: (0, 0)),
    ],
    out_shape=[
        jax.ShapeDtypeStruct((N, 1), jnp.int32),
        jax.ShapeDtypeStruct((1, 1), jnp.float32),
    ],
    compiler_params=pltpu.CompilerParams(
        dimension_semantics=("arbitrary",),
    ),
)


def _gather_body(emb_hbm, idx_hbm, out_hbm, idx_v, rows_v, sem):
    c = lax.axis_index("c")
    s = lax.axis_index("s")
    wid = s * SC_CORES + c
    pltpu.sync_copy(idx_hbm.at[pl.ds(wid * IDX_ROWS_PER_WORKER,
                                     IDX_ROWS_PER_WORKER)], idx_v)
    copies = []
    for j in range(IDX_ROWS_PER_WORKER):
        copies.append(pltpu.async_copy(
            emb_hbm.at[idx_v.at[j]],
            rows_v.at[pl.ds(j * IDX_MINOR, IDX_MINOR)],
            sem))
    for cp in copies:
        cp.wait()
    pltpu.sync_copy(rows_v, out_hbm.at[pl.ds(wid * ROWS_PER_WORKER,
                                             ROWS_PER_WORKER)])


_gather_call = pl.kernel(
    _gather_body,
    out_type=jax.ShapeDtypeStruct((N, D), jnp.float32),
    mesh=plsc.VectorSubcoreMesh(core_axis_name="c", subcore_axis_name="s"),
    scratch_types=[
        pltpu.VMEM((IDX_ROWS_PER_WORKER, IDX_MINOR), jnp.int32),
        pltpu.VMEM((ROWS_PER_WORKER, D), jnp.float32),
        pltpu.SemaphoreType.DMA,
    ],
    compiler_params=pltpu.CompilerParams(use_tc_tiling_on_sc=False),
)


def kernel(latents, embedding):
    flat = jnp.transpose(latents, (0, 2, 3, 1)).reshape(-1, D)
    embt2 = jnp.transpose(embedding * jnp.float32(2.0))

    idx, loss_sum = _argmin_call(flat, embt2)

    q = _gather_call(embedding, idx.reshape(N // IDX_MINOR, IDX_MINOR))

    out = jnp.transpose(q.reshape(32, 32, 32, D), (0, 3, 1, 2))
    vq_loss = loss_sum[0, 0] * jnp.float32((1.0 + BETA) / (N * D))
    return out, vq_loss


# speedup vs baseline: 2.4960x; 1.3945x over previous
import functools

import jax
import jax.numpy as jnp
from jax import lax
from jax.experimental import pallas as pl
from jax.experimental.pallas import tpu as pltpu
from jax.experimental.pallas import tpu_sc as plsc

K = 8192
D = 32
N = 32768
BETA = 0.25

RB = 512
KC = 2048
N_CHUNKS = K // KC
G = 64
NG = RB // G
STRIPS = KC // 128

SC_CORES = 2
SC_SUBCORES = 16
SC_WORKERS = SC_CORES * SC_SUBCORES
ROWS_PER_WORKER = N // SC_WORKERS
IDX_MINOR = 128
IDX_ROWS_PER_WORKER = ROWS_PER_WORKER // IDX_MINOR


def _argmin_body(flat_ref, embt2_ref, idx_ref, loss_ref):
    f = flat_ref[...]
    sumf = jnp.sum(f * f, axis=1, keepdims=True)
    lanef = lax.broadcasted_iota(jnp.int32, (1, 128), 1).astype(jnp.float32)
    big = jnp.float32(2.0**30)

    candmin = [jnp.full((G, 128), jnp.inf, jnp.float32) for _ in range(NG)]
    candidx = [jnp.zeros((G, 128), jnp.float32) for _ in range(NG)]
    for j in range(N_CHUNKS):
        m2 = jnp.dot(f, embt2_ref[:, j * KC:(j + 1) * KC],
                     preferred_element_type=jnp.float32)
        for g in range(NG):
            sf = sumf[g * G:(g + 1) * G, :]
            for s in range(STRIPS):
                d = sf - m2[g * G:(g + 1) * G, s * 128:(s + 1) * 128]
                msk = d < candmin[g]
                candmin[g] = jnp.where(msk, d, candmin[g])
                candidx[g] = jnp.where(msk, jnp.float32(j * STRIPS + s),
                                       candidx[g])

    loss = jnp.float32(0.0)
    for g in range(NG):
        gmin = jnp.min(candmin[g], axis=1, keepdims=True)
        idxf = candidx[g] * jnp.float32(128.0) + lanef
        tie = jnp.where(candmin[g] == gmin, idxf, big)
        rowidx = jnp.min(tie, axis=1, keepdims=True)
        idx_ref[g * G:(g + 1) * G, :] = rowidx.astype(jnp.int32)
        loss = loss + jnp.sum(gmin)

    i = pl.program_id(0)

    @pl.when(i == 0)
    def _init():
        loss_ref[...] = jnp.zeros_like(loss_ref)

    loss_ref[...] += loss.reshape(1, 1)


_argmin_call = pl.pallas_call(
    _argmin_body,
    grid=(N // RB,),
    in_specs=[
        pl.BlockSpec((RB, D), lambda i: (i, 0)),
        pl.BlockSpec((D, K), lambda i: (0, 0)),
    ],
    out_specs=[
        pl.BlockSpec((RB, 1), lambda i: (i, 0)),
        pl.BlockSpec((1, 1), lambda i: (0, 0)),
    ],
    out_shape=[
        jax.ShapeDtypeStruct((N, 1), jnp.int32),
        jax.ShapeDtypeStruct((1, 1), jnp.float32),
    ],
    compiler_params=pltpu.CompilerParams(
        dimension_semantics=("arbitrary",),
    ),
)


def _gather_body(emb_hbm, idx_hbm, out_hbm, idx_v, rows_v, sem):
    c = lax.axis_index("c")
    s = lax.axis_index("s")
    wid = s * SC_CORES + c
    pltpu.sync_copy(idx_hbm.at[pl.ds(wid * IDX_ROWS_PER_WORKER,
                                     IDX_ROWS_PER_WORKER)], idx_v)
    copies = []
    for j in range(IDX_ROWS_PER_WORKER):
        copies.append(pltpu.async_copy(
            emb_hbm.at[idx_v.at[j]],
            rows_v.at[pl.ds(j * IDX_MINOR, IDX_MINOR)],
            sem))
    for cp in copies:
        cp.wait()
    pltpu.sync_copy(rows_v, out_hbm.at[pl.ds(wid * ROWS_PER_WORKER,
                                             ROWS_PER_WORKER)])


_gather_call = pl.kernel(
    _gather_body,
    out_type=jax.ShapeDtypeStruct((N, D), jnp.float32),
    mesh=plsc.VectorSubcoreMesh(core_axis_name="c", subcore_axis_name="s"),
    scratch_types=[
        pltpu.VMEM((IDX_ROWS_PER_WORKER, IDX_MINOR), jnp.int32),
        pltpu.VMEM((ROWS_PER_WORKER, D), jnp.float32),
        pltpu.SemaphoreType.DMA,
    ],
    compiler_params=pltpu.CompilerParams(use_tc_tiling_on_sc=False),
)


def kernel(latents, embedding):
    flat = jnp.transpose(latents, (0, 2, 3, 1)).reshape(-1, D)
    embt2 = jnp.transpose(embedding * jnp.float32(2.0))

    idx, loss_sum = _argmin_call(flat, embt2)

    q = _gather_call(embedding, idx.reshape(N // IDX_MINOR, IDX_MINOR))

    out = jnp.transpose(q.reshape(32, 32, 32, D), (0, 3, 1, 2))
    vq_loss = loss_sum[0, 0] * jnp.float32((1.0 + BETA) / (N * D))
    return out, vq_loss
